# baseline (speedup 1.0000x reference)
import functools

import jax
import jax.numpy as jnp
import numpy as np
from jax.experimental import pallas as pl
from jax.experimental.pallas import tpu as pltpu
from jax.experimental.pallas import tpu_sc as plsc

HIDDEN = 2048
NUM_HEADS = 16
HEAD_DIM = 128
S = 2048
LOCAL_WINDOW = 128
STRIDE = 4
QBLK = 512
NQ = S // QBLK
HALF = QBLK // 2
WIN = HALF + 2 * LOCAL_WINDOW
NBLK = 512
NWB = HIDDEN // NBLK
NEG = jnp.finfo(jnp.float32).min

NC, NS, L = 2, 16, 16
NW = NC * NS
ROWS_W = S // NW
RCH = 8
NCH = ROWS_W // RCH
UNROLL = 8


def _density_sc_call(x2d):
    mesh = plsc.VectorSubcoreMesh(
        core_axis_name="c", subcore_axis_name="s", num_cores=NC, num_subcores=NS
    )

    @functools.partial(
        pl.kernel,
        mesh=mesh,
        out_type=jax.ShapeDtypeStruct((NW, L), jnp.float32),
        scratch_types=[
            pltpu.VMEM((RCH, HIDDEN), jnp.float32),
            pltpu.VMEM((L,), jnp.float32),
            pltpu.SemaphoreType.DMA,
        ],
    )
    def density_kernel(x_hbm, out_hbm, buf, accv, sem):
        wid = jax.lax.axis_index("s") * NC + jax.lax.axis_index("c")
        base = wid * ROWS_W
        ones = jnp.full((L,), 1.0, jnp.float32)
        zeros = jnp.zeros((L,), jnp.float32)

        def chunk_body(c, acc):
            pltpu.async_copy(
                x_hbm.at[pl.ds(base + c * RCH, RCH)], buf, sem
            ).wait()

            def body(i, acc):
                o = i * (L * UNROLL)
                for r in range(RCH):
                    for u in range(UNROLL):
                        v = buf[r, pl.ds(o + u * L, L)]
                        acc = acc + jnp.where(jnp.abs(v) > 0.1, ones, zeros)
                return acc

            return jax.lax.fori_loop(0, HIDDEN // (L * UNROLL), body, acc)

        acc = jax.lax.fori_loop(0, NCH, chunk_body, jnp.zeros((L,), jnp.float32))
        accv[...] = acc
        pltpu.sync_copy(accv, out_hbm.at[wid])

    return density_kernel(x2d)


def _qkv_kernel(x_ref, wq_ref, wk_ref, wv_ref, b_ref, out_ref):
    n = pl.program_id(0)
    g = n // NWB

    def dot(w_ref):
        return jax.lax.dot_general(
            x_ref[...], w_ref[...], (((1,), (1,)), ((), ())),
            preferred_element_type=jnp.float32,
        ) + b_ref[...]

    @pl.when(g == 0)
    def _():
        out_ref[...] = dot(wq_ref)

    @pl.when(g == 1)
    def _():
        out_ref[...] = dot(wk_ref)

    @pl.when(g == 2)
    def _():
        out_ref[...] = dot(wv_ref)


def _attn_kernel(us_ref, q_ref, k_ref, v_ref, out_ref, bias_ref):
    h = pl.program_id(0)
    qi = pl.program_id(1)
    use_strided = us_ref[0] > 0
    scale = 1.0 / np.sqrt(HEAD_DIM)
    q = q_ref[...] * scale

    @pl.when(use_strided)
    def _():
        s = jax.lax.dot_general(
            q, k_ref[...], (((1,), (1,)), ((), ())),
            preferred_element_type=jnp.float32,
        )
        row = qi * QBLK + jax.lax.broadcasted_iota(jnp.int32, (QBLK, S), 0)
        col = jax.lax.broadcasted_iota(jnp.int32, (QBLK, S), 1)
        diff = row - col
        m = (
            (jnp.abs(diff) <= LOCAL_WINDOW)
            | (row == 0)
            | (col == 0)
            | (jax.lax.rem(diff, STRIDE) == 0)
        )
        s = jnp.where(m, s, NEG)
        s = s - jnp.max(s, axis=1, keepdims=True)
        e = jnp.exp(s)
        p = e / jnp.sum(e, axis=1, keepdims=True)
        out_ref[...] = jnp.dot(p, v_ref[...], preferred_element_type=jnp.float32)

    @pl.when(jnp.logical_not(use_strided))
    def _():
        def half(sub):
            rbase = qi * QBLK + sub * HALF
            wstart = pl.multiple_of(
                jnp.maximum(
                    jnp.minimum(rbase - LOCAL_WINDOW, S - WIN), 0
                ),
                LOCAL_WINDOW,
            )
            brow = qi * QBLK + sub * HALF

            @pl.when(h == 0)
            def _():
                off = rbase - wstart
                d0 = off + jax.lax.broadcasted_iota(
                    jnp.int32, (HALF, WIN), 0
                ) - jax.lax.broadcasted_iota(jnp.int32, (HALF, WIN), 1)
                colz = jax.lax.broadcasted_iota(jnp.int32, (HALF, WIN), 1) == 0
                m1 = (jnp.abs(d0) <= LOCAL_WINDOW) | (colz & (wstart == 0))
                bias_ref[pl.ds(brow, HALF), :] = jnp.where(m1, 0.0, NEG)

            qh = q[sub * HALF : sub * HALF + HALF, :]
            kw = k_ref[pl.ds(wstart, WIN), :]
            s1 = jax.lax.dot_general(
                qh, kw, (((1,), (1,)), ((), ())),
                preferred_element_type=jnp.float32,
            ) + bias_ref[pl.ds(brow, HALF), :]
            k0 = k_ref[0:1, :]
            s0 = jax.lax.dot_general(
                qh, k0, (((1,), (1,)), ((), ())),
                preferred_element_type=jnp.float32,
            )
            s0 = jnp.where(wstart > 0, s0, NEG)
            e1 = jnp.exp(s1)
            e0 = jnp.exp(s0)
            denom = jnp.sum(e1, axis=1, keepdims=True) + e0
            vw = v_ref[pl.ds(wstart, WIN), :]
            v0 = v_ref[0:1, :]
            o = jnp.dot(e1, vw, preferred_element_type=jnp.float32) + jnp.dot(
                e0, v0, preferred_element_type=jnp.float32
            )
            return o / denom

        out_ref[0:HALF, :] = half(0)
        out_ref[HALF:QBLK, :] = half(1)

        @pl.when(qi == 0)
        def _():
            sr = jax.lax.dot_general(
                q[0:1, :], k_ref[...], (((1,), (1,)), ((), ())),
                preferred_element_type=jnp.float32,
            )
            mr = jnp.max(sr, axis=1, keepdims=True)
            pr = jnp.exp(sr - mr)
            orow = jnp.dot(
                pr, v_ref[...], preferred_element_type=jnp.float32
            ) / jnp.sum(pr, axis=1, keepdims=True)
            out_ref[0:1, :] = orow


def _proj_kernel(x_ref, w_ref, b_ref, out_ref):
    out_ref[...] = (
        jax.lax.dot_general(
            x_ref[...], w_ref[...], (((1,), (1,)), ((), ())),
            preferred_element_type=jnp.float32,
        )
        + b_ref[...]
    )


def kernel(hidden_states, Wq, bq, Wk, bk, Wv, bv, Wo, bo):
    x = hidden_states[0]
    cntrows = _density_sc_call(x)

    bcat = jnp.concatenate([bq, bk, bv])[None, :]

    n_out = 3 * HIDDEN
    qkv = pl.pallas_call(
        _qkv_kernel,
        grid=(n_out // NBLK,),
        in_specs=[
            pl.BlockSpec((S, HIDDEN), lambda n: (0, 0)),
            pl.BlockSpec((NBLK, HIDDEN), lambda n: (jnp.minimum(n, NWB - 1), 0)),
            pl.BlockSpec(
                (NBLK, HIDDEN),
                lambda n: (jnp.clip(n - NWB, 0, NWB - 1), 0),
            ),
            pl.BlockSpec(
                (NBLK, HIDDEN),
                lambda n: (jnp.clip(n - 2 * NWB, 0, NWB - 1), 0),
            ),
            pl.BlockSpec((1, NBLK), lambda n: (0, n)),
        ],
        out_specs=pl.BlockSpec((S, NBLK), lambda n: (0, n)),
        out_shape=jax.ShapeDtypeStruct((S, n_out), jnp.float32),
    )(x, Wq, Wk, Wv, bcat)

    density = cntrows.sum() / (S * HIDDEN)
    us = (density <= 0.5).astype(jnp.int32).reshape(1)

    attn_out = pl.pallas_call(
        _attn_kernel,
        grid=(NUM_HEADS, NQ),
        in_specs=[
            pl.BlockSpec(memory_space=pltpu.SMEM),
            pl.BlockSpec((QBLK, HEAD_DIM), lambda h, i: (i, h)),
            pl.BlockSpec((S, HEAD_DIM), lambda h, i: (0, NUM_HEADS + h)),
            pl.BlockSpec((S, HEAD_DIM), lambda h, i: (0, 2 * NUM_HEADS + h)),
        ],
        out_specs=pl.BlockSpec((QBLK, HEAD_DIM), lambda h, i: (i, h)),
        out_shape=jax.ShapeDtypeStruct((S, HIDDEN), jnp.float32),
        scratch_shapes=[pltpu.VMEM((S, WIN), jnp.float32)],
    )(us, qkv, qkv, qkv)

    out = pl.pallas_call(
        _proj_kernel,
        grid=(HIDDEN // NBLK,),
        in_specs=[
            pl.BlockSpec((S, HIDDEN), lambda n: (0, 0)),
            pl.BlockSpec((NBLK, HIDDEN), lambda n: (n, 0)),
            pl.BlockSpec((1, NBLK), lambda n: (0, n)),
        ],
        out_specs=pl.BlockSpec((S, NBLK), lambda n: (0, n)),
        out_shape=jax.ShapeDtypeStruct((S, HIDDEN), jnp.float32),
    )(attn_out, Wo, bo[None, :])

    return out[None]


# speedup vs baseline: 1.0222x; 1.0222x over previous
import functools

import jax
import jax.numpy as jnp
import numpy as np
from jax.experimental import pallas as pl
from jax.experimental.pallas import tpu as pltpu
from jax.experimental.pallas import tpu_sc as plsc

HIDDEN = 2048
NUM_HEADS = 16
HEAD_DIM = 128
S = 2048
LOCAL_WINDOW = 128
STRIDE = 4
QBLK = 512
NQ = S // QBLK
WIN = QBLK + 2 * LOCAL_WINDOW
NBLK = 512
NWB = HIDDEN // NBLK
NEG = jnp.finfo(jnp.float32).min

NC, NS, L = 2, 16, 16
NW = NC * NS
ROWS_W = S // NW
RCH = 16
NCH = ROWS_W // RCH
UNROLL = 8


def _density_sc_call(x2d):
    mesh = plsc.VectorSubcoreMesh(
        core_axis_name="c", subcore_axis_name="s", num_cores=NC, num_subcores=NS
    )

    @functools.partial(
        pl.kernel,
        mesh=mesh,
        out_type=jax.ShapeDtypeStruct((NW, L), jnp.float32),
        scratch_types=[
            pltpu.VMEM((RCH, HIDDEN), jnp.float32),
            pltpu.VMEM((RCH, HIDDEN), jnp.float32),
            pltpu.VMEM((L,), jnp.float32),
            pltpu.SemaphoreType.DMA,
            pltpu.SemaphoreType.DMA,
        ],
    )
    def density_kernel(x_hbm, out_hbm, buf0, buf1, accv, sem0, sem1):
        wid = jax.lax.axis_index("s") * NC + jax.lax.axis_index("c")
        base = wid * ROWS_W
        ones = jnp.full((L,), 1.0, jnp.float32)
        zeros = jnp.zeros((L,), jnp.float32)
        bufs = (buf0, buf1)
        sems = (sem0, sem1)

        cps = [None] * NCH
        cps[0] = pltpu.async_copy(x_hbm.at[pl.ds(base, RCH)], buf0, sem0)
        acc = jnp.zeros((L,), jnp.float32)
        for c in range(NCH):
            if c + 1 < NCH:
                cps[c + 1] = pltpu.async_copy(
                    x_hbm.at[pl.ds(base + (c + 1) * RCH, RCH)],
                    bufs[(c + 1) % 2],
                    sems[(c + 1) % 2],
                )
            cps[c].wait()
            buf = bufs[c % 2]

            def body(i, acc, buf=buf):
                o = i * (L * UNROLL)
                for r in range(RCH):
                    for u in range(UNROLL):
                        v = buf[r, pl.ds(o + u * L, L)]
                        acc = acc + jnp.where(jnp.abs(v) > 0.1, ones, zeros)
                return acc

            acc = jax.lax.fori_loop(0, HIDDEN // (L * UNROLL), body, acc)
        accv[...] = acc
        pltpu.sync_copy(accv, out_hbm.at[wid])

    return density_kernel(x2d)


def _qkv_kernel(x_ref, wq_ref, wk_ref, wv_ref, b_ref, out_ref):
    n = pl.program_id(0)
    g = n // NWB

    def dot(w_ref):
        return jax.lax.dot_general(
            x_ref[...], w_ref[...], (((1,), (1,)), ((), ())),
            preferred_element_type=jnp.float32,
        ) + b_ref[...]

    @pl.when(g == 0)
    def _():
        out_ref[...] = dot(wq_ref)

    @pl.when(g == 1)
    def _():
        out_ref[...] = dot(wk_ref)

    @pl.when(g == 2)
    def _():
        out_ref[...] = dot(wv_ref)


def _attn_kernel(us_ref, q_ref, k_ref, v_ref, out_ref, bias_ref):
    h = pl.program_id(0)
    qi = pl.program_id(1)
    use_strided = us_ref[0] > 0
    scale = 1.0 / np.sqrt(HEAD_DIM)
    q = q_ref[...] * scale

    @pl.when(use_strided)
    def _():
        s = jax.lax.dot_general(
            q, k_ref[...], (((1,), (1,)), ((), ())),
            preferred_element_type=jnp.float32,
        )
        row = qi * QBLK + jax.lax.broadcasted_iota(jnp.int32, (QBLK, S), 0)
        col = jax.lax.broadcasted_iota(jnp.int32, (QBLK, S), 1)
        diff = row - col
        m = (
            (jnp.abs(diff) <= LOCAL_WINDOW)
            | (row == 0)
            | (col == 0)
            | (jax.lax.rem(diff, STRIDE) == 0)
        )
        s = jnp.where(m, s, NEG)
        s = s - jnp.max(s, axis=1, keepdims=True)
        e = jnp.exp(s)
        p = e / jnp.sum(e, axis=1, keepdims=True)
        out_ref[...] = jnp.dot(p, v_ref[...], preferred_element_type=jnp.float32)

    @pl.when(jnp.logical_not(use_strided))
    def _():
        wstart = pl.multiple_of(
            jnp.maximum(jnp.minimum(qi * QBLK - LOCAL_WINDOW, S - WIN), 0),
            LOCAL_WINDOW,
        )

        @pl.when(h == 0)
        def _():
            off = qi * QBLK - wstart
            d0 = off + jax.lax.broadcasted_iota(
                jnp.int32, (QBLK, WIN), 0
            ) - jax.lax.broadcasted_iota(jnp.int32, (QBLK, WIN), 1)
            colz = jax.lax.broadcasted_iota(jnp.int32, (QBLK, WIN), 1) == 0
            m1 = (jnp.abs(d0) <= LOCAL_WINDOW) | (colz & (wstart == 0))
            bias_ref[pl.ds(qi * QBLK, QBLK), :] = jnp.where(m1, 0.0, NEG)

        kw = k_ref[pl.ds(wstart, WIN), :]
        s1 = jax.lax.dot_general(
            q, kw, (((1,), (1,)), ((), ())), preferred_element_type=jnp.float32
        ) + bias_ref[pl.ds(qi * QBLK, QBLK), :]
        k0 = k_ref[0:1, :]
        s0 = jax.lax.dot_general(
            q, k0, (((1,), (1,)), ((), ())), preferred_element_type=jnp.float32
        )
        s0 = jnp.where(wstart > 0, s0, NEG)
        e1 = jnp.exp(s1)
        e0 = jnp.exp(s0)
        denom = jnp.sum(e1, axis=1, keepdims=True) + e0
        vw = v_ref[pl.ds(wstart, WIN), :]
        v0 = v_ref[0:1, :]
        o = jnp.dot(e1, vw, preferred_element_type=jnp.float32) + jnp.dot(
            e0, v0, preferred_element_type=jnp.float32
        )
        o = o / denom
        out_ref[...] = o

        @pl.when(qi == 0)
        def _():
            sr = jax.lax.dot_general(
                q[0:1, :], k_ref[...], (((1,), (1,)), ((), ())),
                preferred_element_type=jnp.float32,
            )
            mr = jnp.max(sr, axis=1, keepdims=True)
            pr = jnp.exp(sr - mr)
            orow = jnp.dot(
                pr, v_ref[...], preferred_element_type=jnp.float32
            ) / jnp.sum(pr, axis=1, keepdims=True)
            out_ref[0:1, :] = orow


def _proj_kernel(x_ref, w_ref, b_ref, out_ref):
    out_ref[...] = (
        jax.lax.dot_general(
            x_ref[...], w_ref[...], (((1,), (1,)), ((), ())),
            preferred_element_type=jnp.float32,
        )
        + b_ref[...]
    )


def kernel(hidden_states, Wq, bq, Wk, bk, Wv, bv, Wo, bo):
    x = hidden_states[0]
    cntrows = _density_sc_call(x)

    bcat = jnp.concatenate([bq, bk, bv])[None, :]

    n_out = 3 * HIDDEN
    qkv = pl.pallas_call(
        _qkv_kernel,
        grid=(n_out // NBLK,),
        in_specs=[
            pl.BlockSpec((S, HIDDEN), lambda n: (0, 0)),
            pl.BlockSpec((NBLK, HIDDEN), lambda n: (jnp.minimum(n, NWB - 1), 0)),
            pl.BlockSpec(
                (NBLK, HIDDEN),
                lambda n: (jnp.clip(n - NWB, 0, NWB - 1), 0),
            ),
            pl.BlockSpec(
                (NBLK, HIDDEN),
                lambda n: (jnp.clip(n - 2 * NWB, 0, NWB - 1), 0),
            ),
            pl.BlockSpec((1, NBLK), lambda n: (0, n)),
        ],
        out_specs=pl.BlockSpec((S, NBLK), lambda n: (0, n)),
        out_shape=jax.ShapeDtypeStruct((S, n_out), jnp.float32),
    )(x, Wq, Wk, Wv, bcat)

    density = cntrows.sum() / (S * HIDDEN)
    us = (density <= 0.5).astype(jnp.int32).reshape(1)

    attn_out = pl.pallas_call(
        _attn_kernel,
        grid=(NUM_HEADS, NQ),
        in_specs=[
            pl.BlockSpec(memory_space=pltpu.SMEM),
            pl.BlockSpec((QBLK, HEAD_DIM), lambda h, i: (i, h)),
            pl.BlockSpec((S, HEAD_DIM), lambda h, i: (0, NUM_HEADS + h)),
            pl.BlockSpec((S, HEAD_DIM), lambda h, i: (0, 2 * NUM_HEADS + h)),
        ],
        out_specs=pl.BlockSpec((QBLK, HEAD_DIM), lambda h, i: (i, h)),
        out_shape=jax.ShapeDtypeStruct((S, HIDDEN), jnp.float32),
        scratch_shapes=[pltpu.VMEM((S, WIN), jnp.float32)],
    )(us, qkv, qkv, qkv)

    out = pl.pallas_call(
        _proj_kernel,
        grid=(HIDDEN // NBLK,),
        in_specs=[
            pl.BlockSpec((S, HIDDEN), lambda n: (0, 0)),
            pl.BlockSpec((NBLK, HIDDEN), lambda n: (n, 0)),
            pl.BlockSpec((1, NBLK), lambda n: (0, n)),
        ],
        out_specs=pl.BlockSpec((S, NBLK), lambda n: (0, n)),
        out_shape=jax.ShapeDtypeStruct((S, HIDDEN), jnp.float32),
    )(attn_out, Wo, bo[None, :])

    return out[None]
